# baseline (speedup 1.0000x reference)
import functools

import numpy as np
import jax
import jax.numpy as jnp
from jax import lax
from jax.experimental import pallas as pl
from jax.experimental.pallas import tpu as pltpu
from jax.experimental.pallas import tpu_sc as plsc

_NC = 2
_NS = 16
_NW = _NC * _NS


@functools.lru_cache(maxsize=None)
def _pe_const(seq_len: int, d_model: int):
    pos = np.arange(seq_len)[:, None].astype(np.float64)
    i = np.arange(d_model)[None, :]
    angle_rates = 1.0 / np.power(10000.0, (2 * (i // 2)) / float(d_model))
    angles = pos * angle_rates
    pe = np.zeros((seq_len, d_model), dtype=np.float32)
    pe[:, 0::2] = np.sin(angles[:, 0::2])
    pe[:, 1::2] = np.cos(angles[:, 1::2])
    return jnp.asarray(pe)


@functools.lru_cache(maxsize=None)
def _build(B: int, S: int, D: int):
    SPW = S // _NW
    NV = D // 16
    CH = 32
    NCH = B * SPW // CH
    HPB = SPW // CH
    JB = 6

    mesh = plsc.VectorSubcoreMesh(core_axis_name="c", subcore_axis_name="s")

    @functools.partial(
        pl.kernel,
        out_type=jax.ShapeDtypeStruct((B, S, D), jnp.float32),
        mesh=mesh,
        scratch_types=[
            pltpu.VMEM((SPW, D), jnp.float32),
            pltpu.VMEM((CH, D), jnp.float32),
            pltpu.VMEM((CH, D), jnp.float32),
            pltpu.VMEM((B * SPW,), jnp.int32),
            pltpu.VMEM((B * SPW + 16,), jnp.int32),
            pltpu.VMEM((3 * D,), jnp.float32),
            pltpu.SemaphoreType.DMA,
            pltpu.SemaphoreType.DMA,
            pltpu.SemaphoreType.DMA,
            pltpu.SemaphoreType.DMA,
        ],
    )
    def emb(seq_hbm, lab_hbm, tok_hbm, segf_hbm, pe_hbm, out_hbm,
            pe_v, rows0, rows1, idx_v, lab_v, seg_v,
            gsem0, gsem1, wsem0, wsem1):
        wid = lax.axis_index("s") * _NC + lax.axis_index("c")
        s0 = wid * SPW
        pltpu.sync_copy(pe_hbm.at[pl.ds(s0, SPW)], pe_v)
        pltpu.sync_copy(segf_hbm, seg_v)
        for b in range(B):
            pltpu.sync_copy(seq_hbm.at[b, pl.ds(s0, SPW)],
                            idx_v.at[pl.ds(b * SPW, SPW)])
            pltpu.sync_copy(lab_hbm.at[b, pl.ds(s0, SPW)],
                            lab_v.at[pl.ds(b * SPW, SPW)])

        rows = (rows0, rows1)
        gsem = (gsem0, gsem1)
        wsem = (wsem0, wsem1)
        gd = [None] * NCH
        wd = [None] * NCH

        def fused_pass(c):
            b, h = divmod(c, HPB)
            buf = rows[c % 2]
            for g in range(CH // 16):
                lab16 = lab_v[pl.ds(c * CH + g * 16, 16)]
                lf16 = lab16.astype(jnp.float32)
                m1_16 = lf16 * (2.0 - lf16)
                m2_16 = lf16 * (lf16 - 1.0) * 0.5

                def jb_body(jb, carry, g=g, m1_16=m1_16, m2_16=m2_16):
                    s0 = [seg_v[pl.ds(0 * D + (jb * JB + u) * 16, 16)]
                          for u in range(JB)]
                    d1 = [seg_v[pl.ds(1 * D + (jb * JB + u) * 16, 16)] - s0[u]
                          for u in range(JB)]
                    d2 = [seg_v[pl.ds(2 * D + (jb * JB + u) * 16, 16)] - s0[u]
                          for u in range(JB)]

                    def tok_body(t, tcarry):
                        ts0, td1, td2, tm1, tm2 = tcarry
                        sp = jnp.broadcast_to(t, (16,))
                        m1 = tm1.at[sp].get(mode="promise_in_bounds")
                        m2 = tm2.at[sp].get(mode="promise_in_bounds")
                        i = g * 16 + t
                        prow = h * CH + i
                        for u in range(JB):
                            sl = pl.ds((jb * JB + u) * 16, 16)
                            seg = ts0[u] + (m1 * td1[u] + m2 * td2[u])
                            buf[i, sl] = (buf[i, sl] + pe_v[prow, sl]) + seg
                        return tcarry

                    plsc.parallel_loop(
                        0, 16, 1, unroll=4,
                        carry=(s0, d1, d2, m1_16, m2_16))(tok_body)
                    return carry

                lax.fori_loop(0, NV // JB, jb_body, 0)

        def start_gather(c):
            gd[c] = pltpu.async_copy(
                tok_hbm.at[idx_v.at[pl.ds(c * CH, CH)]], rows[c % 2],
                gsem[c % 2])

        def start_write(c):
            b, h = divmod(c, HPB)
            wd[c] = pltpu.async_copy(
                rows[c % 2], out_hbm.at[b, pl.ds(s0 + h * CH, CH)], wsem[c % 2])

        start_gather(0)
        for c in range(NCH):
            if c + 1 < NCH:
                if c - 1 >= 0:
                    wd[c - 1].wait()
                start_gather(c + 1)
            gd[c].wait()
            fused_pass(c)
            start_write(c)
        wd[NCH - 2].wait()
        wd[NCH - 1].wait()

    return emb


def kernel(sequence, segment_label, token_table, segment_table):
    B, S = sequence.shape
    D = token_table.shape[1]
    pe = _pe_const(S, D)
    seq = sequence.astype(jnp.int32)
    lab = segment_label.astype(jnp.int32)
    segf = segment_table.astype(jnp.float32).reshape(-1)
    return _build(B, S, D)(seq, lab, token_table.astype(jnp.float32), segf, pe)


# speedup vs baseline: 1.0934x; 1.0934x over previous
import functools

import numpy as np
import jax
import jax.numpy as jnp
from jax import lax
from jax.experimental import pallas as pl
from jax.experimental.pallas import tpu as pltpu
from jax.experimental.pallas import tpu_sc as plsc

_NC = 2
_NS = 16
_NW = _NC * _NS


@functools.lru_cache(maxsize=None)
def _pe_const(seq_len: int, d_model: int):
    pos = np.arange(seq_len)[:, None].astype(np.float64)
    i = np.arange(d_model)[None, :]
    angle_rates = 1.0 / np.power(10000.0, (2 * (i // 2)) / float(d_model))
    angles = pos * angle_rates
    pe = np.zeros((seq_len, d_model), dtype=np.float32)
    pe[:, 0::2] = np.sin(angles[:, 0::2])
    pe[:, 1::2] = np.cos(angles[:, 1::2])
    return jnp.asarray(pe)


@functools.lru_cache(maxsize=None)
def _build(B: int, S: int, D: int):
    SPW = S // _NW
    NV = D // 16
    CH = 32
    NCH = B * SPW // CH
    HPB = SPW // CH
    JB = 4

    mesh = plsc.VectorSubcoreMesh(core_axis_name="c", subcore_axis_name="s")

    @functools.partial(
        pl.kernel,
        out_type=jax.ShapeDtypeStruct((B, S, D), jnp.float32),
        mesh=mesh,
        scratch_types=[
            pltpu.VMEM((SPW, D), jnp.float32),
            pltpu.VMEM((CH, D), jnp.float32),
            pltpu.VMEM((CH, D), jnp.float32),
            pltpu.VMEM((B * SPW,), jnp.int32),
            pltpu.VMEM((B * SPW + 16,), jnp.int32),
            pltpu.VMEM((3 * D,), jnp.float32),
            pltpu.SemaphoreType.DMA,
            pltpu.SemaphoreType.DMA,
            pltpu.SemaphoreType.DMA,
            pltpu.SemaphoreType.DMA,
        ],
    )
    def emb(seq_hbm, lab_hbm, tok_hbm, segf_hbm, pe_hbm, out_hbm,
            pe_v, rows0, rows1, idx_v, lab_v, seg_v,
            gsem0, gsem1, wsem0, wsem1):
        wid = lax.axis_index("s") * _NC + lax.axis_index("c")
        s0 = wid * SPW
        pltpu.sync_copy(pe_hbm.at[pl.ds(s0, SPW)], pe_v)
        pltpu.sync_copy(segf_hbm, seg_v)
        for b in range(B):
            pltpu.sync_copy(seq_hbm.at[b, pl.ds(s0, SPW)],
                            idx_v.at[pl.ds(b * SPW, SPW)])
            pltpu.sync_copy(lab_hbm.at[b, pl.ds(s0, SPW)],
                            lab_v.at[pl.ds(b * SPW, SPW)])

        rows = (rows0, rows1)
        gsem = (gsem0, gsem1)
        wsem = (wsem0, wsem1)
        gd = [None] * NCH
        wd = [None] * NCH

        def fused_pass(c):
            b, h = divmod(c, HPB)
            buf = rows[c % 2]
            ms = []
            for g in range(CH // 16):
                lab16 = lab_v[pl.ds(c * CH + g * 16, 16)]
                lf16 = lab16.astype(jnp.float32)
                ms.append(lf16 * (2.0 - lf16))
                ms.append(lf16 * (lf16 - 1.0) * 0.5)

            def jb_body(jb, carry):
                s0 = [seg_v[pl.ds(0 * D + (jb * JB + u) * 16, 16)]
                      for u in range(JB)]
                d1 = [seg_v[pl.ds(1 * D + (jb * JB + u) * 16, 16)] - s0[u]
                      for u in range(JB)]
                d2 = [seg_v[pl.ds(2 * D + (jb * JB + u) * 16, 16)] - s0[u]
                      for u in range(JB)]
                for g in range(CH // 16):

                    def tok_body(t, tcarry, g=g):
                        ts0, td1, td2, tm1, tm2 = tcarry
                        sp = jnp.broadcast_to(t, (16,))
                        m1 = tm1.at[sp].get(mode="promise_in_bounds")
                        m2 = tm2.at[sp].get(mode="promise_in_bounds")
                        i = g * 16 + t
                        prow = h * CH + i
                        for u in range(JB):
                            sl = pl.ds((jb * JB + u) * 16, 16)
                            seg = ts0[u] + (m1 * td1[u] + m2 * td2[u])
                            buf[i, sl] = (buf[i, sl] + pe_v[prow, sl]) + seg
                        return tcarry

                    plsc.parallel_loop(
                        0, 16, 1, unroll=4,
                        carry=(s0, d1, d2,
                               carry[2 * g], carry[2 * g + 1]))(tok_body)
                return carry

            lax.fori_loop(0, NV // JB, jb_body, tuple(ms))

        def start_gather(c):
            gd[c] = pltpu.async_copy(
                tok_hbm.at[idx_v.at[pl.ds(c * CH, CH)]], rows[c % 2],
                gsem[c % 2])

        def start_write(c):
            b, h = divmod(c, HPB)
            wd[c] = pltpu.async_copy(
                rows[c % 2], out_hbm.at[b, pl.ds(s0 + h * CH, CH)], wsem[c % 2])

        start_gather(0)
        for c in range(NCH):
            if c + 1 < NCH:
                if c - 1 >= 0:
                    wd[c - 1].wait()
                start_gather(c + 1)
            gd[c].wait()
            fused_pass(c)
            start_write(c)
        wd[NCH - 2].wait()
        wd[NCH - 1].wait()

    return emb


def kernel(sequence, segment_label, token_table, segment_table):
    B, S = sequence.shape
    D = token_table.shape[1]
    pe = _pe_const(S, D)
    seq = sequence.astype(jnp.int32)
    lab = segment_label.astype(jnp.int32)
    segf = segment_table.astype(jnp.float32).reshape(-1)
    return _build(B, S, D)(seq, lab, token_table.astype(jnp.float32), segf, pe)


# speedup vs baseline: 1.1948x; 1.0927x over previous
import functools

import numpy as np
import jax
import jax.numpy as jnp
from jax import lax
from jax.experimental import pallas as pl
from jax.experimental.pallas import tpu as pltpu
from jax.experimental.pallas import tpu_sc as plsc

_NC = 2
_NS = 16
_NW = _NC * _NS


@functools.lru_cache(maxsize=None)
def _pe_const(seq_len: int, d_model: int):
    pos = np.arange(seq_len)[:, None].astype(np.float64)
    i = np.arange(d_model)[None, :]
    angle_rates = 1.0 / np.power(10000.0, (2 * (i // 2)) / float(d_model))
    angles = pos * angle_rates
    pe = np.zeros((seq_len, d_model), dtype=np.float32)
    pe[:, 0::2] = np.sin(angles[:, 0::2])
    pe[:, 1::2] = np.cos(angles[:, 1::2])
    return jnp.asarray(pe)


@functools.lru_cache(maxsize=None)
def _build(B: int, S: int, D: int):
    SPW = S // _NW
    NV = D // 16
    CH = 32
    NCH = B * SPW // CH
    HPB = SPW // CH
    JB = 4

    mesh = plsc.VectorSubcoreMesh(core_axis_name="c", subcore_axis_name="s")

    @functools.partial(
        pl.kernel,
        out_type=jax.ShapeDtypeStruct((B, S, D), jnp.float32),
        mesh=mesh,
        scratch_types=[
            pltpu.VMEM((SPW, D), jnp.float32),
            pltpu.VMEM((CH, D), jnp.float32),
            pltpu.VMEM((CH, D), jnp.float32),
            pltpu.VMEM((CH, D), jnp.float32),
            pltpu.VMEM((B * SPW,), jnp.int32),
            pltpu.VMEM((B * SPW + 16,), jnp.int32),
            pltpu.VMEM((3 * D,), jnp.float32),
            pltpu.SemaphoreType.DMA,
            pltpu.SemaphoreType.DMA,
            pltpu.SemaphoreType.DMA,
            pltpu.SemaphoreType.DMA,
            pltpu.SemaphoreType.DMA,
            pltpu.SemaphoreType.DMA,
        ],
    )
    def emb(seq_hbm, lab_hbm, tok_hbm, segf_hbm, pe_hbm, out_hbm,
            pe_v, rows0, rows1, rows2, idx_v, lab_v, seg_v,
            gsem0, gsem1, gsem2, wsem0, wsem1, wsem2):
        wid = lax.axis_index("s") * _NC + lax.axis_index("c")
        s0 = wid * SPW
        pltpu.sync_copy(pe_hbm.at[pl.ds(s0, SPW)], pe_v)
        pltpu.sync_copy(segf_hbm, seg_v)
        for b in range(B):
            pltpu.sync_copy(seq_hbm.at[b, pl.ds(s0, SPW)],
                            idx_v.at[pl.ds(b * SPW, SPW)])
            pltpu.sync_copy(lab_hbm.at[b, pl.ds(s0, SPW)],
                            lab_v.at[pl.ds(b * SPW, SPW)])

        rows = (rows0, rows1, rows2)
        gsem = (gsem0, gsem1, gsem2)
        wsem = (wsem0, wsem1, wsem2)
        NB = len(rows)
        gd = [None] * NCH
        wd = [None] * NCH

        def fused_pass(c):
            b, h = divmod(c, HPB)
            buf = rows[c % NB]
            ms = []
            for g in range(CH // 16):
                lab16 = lab_v[pl.ds(c * CH + g * 16, 16)]
                lf16 = lab16.astype(jnp.float32)
                ms.append(lf16 * (2.0 - lf16))
                ms.append(lf16 * (lf16 - 1.0) * 0.5)

            def jb_body(jb, carry):
                s0 = [seg_v[pl.ds(0 * D + (jb * JB + u) * 16, 16)]
                      for u in range(JB)]
                d1 = [seg_v[pl.ds(1 * D + (jb * JB + u) * 16, 16)] - s0[u]
                      for u in range(JB)]
                d2 = [seg_v[pl.ds(2 * D + (jb * JB + u) * 16, 16)] - s0[u]
                      for u in range(JB)]
                for g in range(CH // 16):

                    def tok_body(t, tcarry, g=g):
                        ts0, td1, td2, tm1, tm2 = tcarry
                        sp = jnp.broadcast_to(t, (16,))
                        m1 = tm1.at[sp].get(mode="promise_in_bounds")
                        m2 = tm2.at[sp].get(mode="promise_in_bounds")
                        i = g * 16 + t
                        prow = h * CH + i
                        for u in range(JB):
                            sl = pl.ds((jb * JB + u) * 16, 16)
                            seg = ts0[u] + (m1 * td1[u] + m2 * td2[u])
                            buf[i, sl] = (buf[i, sl] + pe_v[prow, sl]) + seg
                        return tcarry

                    plsc.parallel_loop(
                        0, 16, 1, unroll=4,
                        carry=(s0, d1, d2,
                               carry[2 * g], carry[2 * g + 1]))(tok_body)
                return carry

            lax.fori_loop(0, NV // JB, jb_body, tuple(ms))

        def start_gather(c):
            gd[c] = pltpu.async_copy(
                tok_hbm.at[idx_v.at[pl.ds(c * CH, CH)]], rows[c % NB],
                gsem[c % NB])

        def start_write(c):
            b, h = divmod(c, HPB)
            wd[c] = pltpu.async_copy(
                rows[c % NB], out_hbm.at[b, pl.ds(s0 + h * CH, CH)],
                wsem[c % NB])

        start_gather(0)
        start_gather(1)
        for c in range(NCH):
            gd[c].wait()
            fused_pass(c)
            start_write(c)
            if c + 2 < NCH:
                if c - 1 >= 0:
                    wd[c - 1].wait()
                start_gather(c + 2)
        wd[NCH - 3].wait()
        wd[NCH - 2].wait()
        wd[NCH - 1].wait()

    return emb


def kernel(sequence, segment_label, token_table, segment_table):
    B, S = sequence.shape
    D = token_table.shape[1]
    pe = _pe_const(S, D)
    seq = sequence.astype(jnp.int32)
    lab = segment_label.astype(jnp.int32)
    segf = segment_table.astype(jnp.float32).reshape(-1)
    return _build(B, S, D)(seq, lab, token_table.astype(jnp.float32), segf, pe)


# speedup vs baseline: 1.2874x; 1.0775x over previous
import functools

import numpy as np
import jax
import jax.numpy as jnp
from jax import lax
from jax.experimental import pallas as pl
from jax.experimental.pallas import tpu as pltpu
from jax.experimental.pallas import tpu_sc as plsc

_NC = 2
_NS = 16
_NW = _NC * _NS


@functools.lru_cache(maxsize=None)
def _pe_const(seq_len: int, d_model: int):
    pos = np.arange(seq_len)[:, None].astype(np.float64)
    i = np.arange(d_model)[None, :]
    angle_rates = 1.0 / np.power(10000.0, (2 * (i // 2)) / float(d_model))
    angles = pos * angle_rates
    pe = np.zeros((seq_len, d_model), dtype=np.float32)
    pe[:, 0::2] = np.sin(angles[:, 0::2])
    pe[:, 1::2] = np.cos(angles[:, 1::2])
    return jnp.asarray(pe)


@functools.lru_cache(maxsize=None)
def _build(B: int, S: int, D: int):
    SPW = S // _NW
    NV = D // 16
    CH = 32
    NCH = B * SPW // CH
    HPB = SPW // CH
    JB = 4

    mesh = plsc.VectorSubcoreMesh(core_axis_name="c", subcore_axis_name="s")

    @functools.partial(
        pl.kernel,
        out_type=jax.ShapeDtypeStruct((B, S, D), jnp.float32),
        mesh=mesh,
        scratch_types=[
            pltpu.VMEM((SPW, D), jnp.float32),
            pltpu.VMEM((CH, D), jnp.float32),
            pltpu.VMEM((CH, D), jnp.float32),
            pltpu.VMEM((CH, D), jnp.float32),
            pltpu.VMEM((B * SPW,), jnp.int32),
            pltpu.VMEM((B * SPW,), jnp.int32),
            pltpu.VMEM((3 * D,), jnp.float32),
            pltpu.SemaphoreType.DMA,
            pltpu.SemaphoreType.DMA,
            pltpu.SemaphoreType.DMA,
            pltpu.SemaphoreType.DMA,
            pltpu.SemaphoreType.DMA,
            pltpu.SemaphoreType.DMA,
            pltpu.SemaphoreType.DMA,
            pltpu.SemaphoreType.DMA,
        ],
    )
    def emb(seq_hbm, lab_hbm, tok_hbm, segf_hbm, pe_hbm, out_hbm,
            pe_v, rows0, rows1, rows2, idx_v, lab_v, seg_v,
            gsem0, gsem1, gsem2, wsem0, wsem1, wsem2, psem0, psem1):
        wid = lax.axis_index("s") * _NC + lax.axis_index("c")
        s0 = wid * SPW
        idx_ds = [pltpu.async_copy(seq_hbm.at[b, pl.ds(s0, SPW)],
                                   idx_v.at[pl.ds(b * SPW, SPW)], psem0)
                  for b in range(B)]
        lab_ds = [pltpu.async_copy(lab_hbm.at[b, pl.ds(s0, SPW)],
                                   lab_v.at[pl.ds(b * SPW, SPW)], psem1)
                  for b in range(B)]
        pe_d = pltpu.async_copy(pe_hbm.at[pl.ds(s0, SPW)], pe_v, psem1)
        seg_d = pltpu.async_copy(segf_hbm, seg_v, psem1)

        rows = (rows0, rows1, rows2)
        gsem = (gsem0, gsem1, gsem2)
        wsem = (wsem0, wsem1, wsem2)
        NB = len(rows)
        gd = [None] * NCH
        wd = [None] * NCH

        def fused_pass(c):
            b, h = divmod(c, HPB)
            buf = rows[c % NB]
            ms = []
            for g in range(CH // 16):
                lab16 = lab_v[pl.ds(c * CH + g * 16, 16)]
                lf16 = lab16.astype(jnp.float32)
                ms.append(lf16 * (2.0 - lf16))
                ms.append(lf16 * (lf16 - 1.0) * 0.5)

            def jb_body(jb, carry):
                s0 = [seg_v[pl.ds(0 * D + (jb * JB + u) * 16, 16)]
                      for u in range(JB)]
                d1 = [seg_v[pl.ds(1 * D + (jb * JB + u) * 16, 16)] - s0[u]
                      for u in range(JB)]
                d2 = [seg_v[pl.ds(2 * D + (jb * JB + u) * 16, 16)] - s0[u]
                      for u in range(JB)]
                for g in range(CH // 16):

                    def tok_body(t, tcarry, g=g):
                        ts0, td1, td2, tm1, tm2 = tcarry
                        sp = jnp.broadcast_to(t, (16,))
                        m1 = tm1.at[sp].get(mode="promise_in_bounds")
                        m2 = tm2.at[sp].get(mode="promise_in_bounds")
                        i = g * 16 + t
                        prow = h * CH + i
                        for u in range(JB):
                            sl = pl.ds((jb * JB + u) * 16, 16)
                            seg = ts0[u] + (m1 * td1[u] + m2 * td2[u])
                            buf[i, sl] = (buf[i, sl] + pe_v[prow, sl]) + seg
                        return tcarry

                    plsc.parallel_loop(
                        0, 16, 1, unroll=4,
                        carry=(s0, d1, d2,
                               carry[2 * g], carry[2 * g + 1]))(tok_body)
                return carry

            lax.fori_loop(0, NV // JB, jb_body, tuple(ms))

        def start_gather(c):
            gd[c] = pltpu.async_copy(
                tok_hbm.at[idx_v.at[pl.ds(c * CH, CH)]], rows[c % NB],
                gsem[c % NB])

        def start_write(c):
            b, h = divmod(c, HPB)
            wd[c] = pltpu.async_copy(
                rows[c % NB], out_hbm.at[b, pl.ds(s0 + h * CH, CH)],
                wsem[c % NB])

        for d in idx_ds:
            d.wait()
        start_gather(0)
        start_gather(1)
        for d in lab_ds:
            d.wait()
        pe_d.wait()
        seg_d.wait()
        for c in range(NCH):
            gd[c].wait()
            fused_pass(c)
            start_write(c)
            if c + 2 < NCH:
                if c - 1 >= 0:
                    wd[c - 1].wait()
                start_gather(c + 2)
        wd[NCH - 3].wait()
        wd[NCH - 2].wait()
        wd[NCH - 1].wait()

    return emb


def kernel(sequence, segment_label, token_table, segment_table):
    B, S = sequence.shape
    D = token_table.shape[1]
    pe = _pe_const(S, D)
    seq = sequence.astype(jnp.int32)
    lab = segment_label.astype(jnp.int32)
    segf = segment_table.astype(jnp.float32).reshape(-1)
    return _build(B, S, D)(seq, lab, token_table.astype(jnp.float32), segf, pe)
